# baseline (speedup 1.0000x reference)
import jax
import jax.numpy as jnp
from jax import lax
from jax.experimental import pallas as pl
from jax.experimental.pallas import tpu as pltpu
from jax.experimental.pallas import tpu_sc as plsc

_NC = 2
_NS = 16
_NW = _NC * _NS
_L = 16
_W = 128


def _build_sampler(M, B, d_obs, d_act):
    assert B % _NW == 0
    bpw = B // _NW
    assert bpw % _L == 0 and bpw <= 128
    obs_per_row = _W // d_obs
    act_per_row = _W // d_act
    obs_sh = obs_per_row.bit_length() - 1
    act_sh = act_per_row.bit_length() - 1
    nobs_off = M * d_obs // _W
    mesh = plsc.VectorSubcoreMesh(core_axis_name="c", subcore_axis_name="s")

    def body(cat_hbm, act_hbm, rew_hbm, dw_hbm, idx_hbm,
             obs_out, act_out, rew_out, nobs_out, done_out,
             idx_v, idxo_v, idxn_v, idxa_v, obs_g, act_g, nobs_g,
             rew_v, dw_v, obs_o, nobs_o, act_o, done_v, sem):
        wid = lax.axis_index("s") * _NC + lax.axis_index("c")
        base = wid * bpw
        pltpu.sync_copy(idx_hbm.at[pl.ds(base, bpw)], idx_v)
        for g in range(bpw // _L):
            s = pl.ds(g * _L, _L)
            kv = idx_v[s]
            ko = lax.shift_right_logical(kv, obs_sh)
            idxo_v[s] = ko
            idxn_v[s] = ko + nobs_off
            idxa_v[s] = lax.shift_right_logical(kv, act_sh)
        copies = [
            pltpu.async_copy(cat_hbm.at[idxo_v], obs_g, sem),
            pltpu.async_copy(cat_hbm.at[idxn_v], nobs_g, sem),
            pltpu.async_copy(act_hbm.at[idxa_v], act_g, sem),
            pltpu.async_copy(rew_hbm.at[idx_v], rew_v, sem),
            pltpu.async_copy(dw_hbm.at[idxo_v], dw_v, sem),
        ]
        for cp in copies:
            cp.wait()
        iota = lax.iota(jnp.int32, _L)
        for g in range(bpw // _L):
            s = pl.ds(g * _L, _L)
            kv = idx_v[s]
            rows = iota + g * _L
            cb = lax.shift_left(lax.bitwise_and(kv, obs_per_row - 1), 5)
            fb = rows * d_obs
            for j in range(d_obs):
                v = plsc.load_gather(obs_g, [rows, cb + j])
                plsc.store_scatter(obs_o, [fb + j], v)
                v2 = plsc.load_gather(nobs_g, [rows, cb + j])
                plsc.store_scatter(nobs_o, [fb + j], v2)
            ca = lax.shift_left(lax.bitwise_and(kv, act_per_row - 1), 3)
            fa = rows * d_act
            for j in range(d_act):
                v = plsc.load_gather(act_g, [rows, ca + j])
                plsc.store_scatter(act_o, [fa + j], v)
            sh = lax.shift_left(lax.bitwise_and(kv, 3), 3)
            done_v[s] = lax.bitwise_and(
                lax.shift_right_logical(dw_v[s], sh), 0xFF)
        pltpu.sync_copy(obs_o, obs_out.at[pl.ds(base * d_obs,
                                                bpw * d_obs)])
        pltpu.sync_copy(nobs_o, nobs_out.at[pl.ds(base * d_obs,
                                                  bpw * d_obs)])
        pltpu.sync_copy(act_o, act_out.at[pl.ds(base * d_act,
                                                bpw * d_act)])
        pltpu.sync_copy(rew_v, rew_out.at[pl.ds(base, bpw)])
        pltpu.sync_copy(done_v, done_out.at[pl.ds(base, bpw)])

    return pl.kernel(
        body,
        out_type=(
            jax.ShapeDtypeStruct((B * d_obs,), jnp.float32),
            jax.ShapeDtypeStruct((B * d_act,), jnp.float32),
            jax.ShapeDtypeStruct((B,), jnp.float32),
            jax.ShapeDtypeStruct((B * d_obs,), jnp.float32),
            jax.ShapeDtypeStruct((B,), jnp.int32),
        ),
        mesh=mesh,
        compiler_params=pltpu.CompilerParams(needs_layout_passes=False),
        scratch_types=[
            pltpu.VMEM((bpw,), jnp.int32),
            pltpu.VMEM((bpw,), jnp.int32),
            pltpu.VMEM((bpw,), jnp.int32),
            pltpu.VMEM((bpw,), jnp.int32),
            pltpu.VMEM((bpw, _W), jnp.float32),
            pltpu.VMEM((bpw, _W), jnp.float32),
            pltpu.VMEM((bpw, _W), jnp.float32),
            pltpu.VMEM((bpw,), jnp.float32),
            pltpu.VMEM((bpw,), jnp.int32),
            pltpu.VMEM((bpw * d_obs,), jnp.float32),
            pltpu.VMEM((bpw * d_obs,), jnp.float32),
            pltpu.VMEM((bpw * d_act,), jnp.float32),
            pltpu.VMEM((bpw,), jnp.int32),
            pltpu.SemaphoreType.DMA,
        ],
    )


def kernel(obs_mem, action_mem, reward_mem, next_obs_mem, done_mem, idx):
    M, d_obs = obs_mem.shape
    d_act = action_mem.shape[1]
    B = idx.shape[0]
    cat2 = jnp.stack([obs_mem, next_obs_mem]).reshape(2 * M * d_obs // _W,
                                                     _W)
    act2 = action_mem.reshape(M * d_act // _W, _W)
    done_words = lax.bitcast_convert_type(
        done_mem.astype(jnp.uint8).reshape(M // 4, 4), jnp.int32)
    sampler = _build_sampler(M, B, d_obs, d_act)
    obs_f, act_f, rew_b, nobs_f, done_i = sampler(
        cat2, act2, reward_mem, done_words, idx)
    return (obs_f.reshape(B, d_obs), act_f.reshape(B, d_act), rew_b,
            nobs_f.reshape(B, d_obs), done_i.astype(jnp.bool_))


# speedup vs baseline: 1.1396x; 1.1396x over previous
import jax
import jax.numpy as jnp
from jax import lax
from jax.experimental import pallas as pl
from jax.experimental.pallas import tpu as pltpu
from jax.experimental.pallas import tpu_sc as plsc

_NC = 2
_NS = 16
_NW = _NC * _NS
_L = 16


def _build_sampler(B, d_obs, d_act):
    assert B % _NW == 0
    bpw = B // _NW
    assert bpw % 8 == 0 and bpw % _L == 0 and bpw <= 128
    mesh = plsc.VectorSubcoreMesh(core_axis_name="c", subcore_axis_name="s")

    def body(obs_hbm, act_hbm, rew_hbm, nobs_hbm, dw_hbm, idx_hbm,
             obs_out, act_out, rew_out, nobs_out, done_out,
             idx_v, idxw_v, obs_v, act_v, rew_v, nobs_v, dw_v, done_v, sem):
        wid = lax.axis_index("s") * _NC + lax.axis_index("c")
        base = wid * bpw
        pltpu.sync_copy(idx_hbm.at[pl.ds(base, bpw)], idx_v)
        for i in range(bpw // _L):
            s = pl.ds(i * _L, _L)
            idxw_v[s] = lax.shift_right_logical(idx_v[s], 2)
        copies = [
            pltpu.async_copy(obs_hbm.at[idx_v], obs_v, sem),
            pltpu.async_copy(act_hbm.at[idx_v], act_v, sem),
            pltpu.async_copy(rew_hbm.at[idx_v], rew_v, sem),
            pltpu.async_copy(nobs_hbm.at[idx_v], nobs_v, sem),
            pltpu.async_copy(dw_hbm.at[idxw_v], dw_v, sem),
        ]
        for cp in copies:
            cp.wait()
        for i in range(bpw // _L):
            s = pl.ds(i * _L, _L)
            sh = lax.shift_left(lax.bitwise_and(idx_v[s], 3), 3)
            done_v[s] = lax.bitwise_and(
                lax.shift_right_logical(dw_v[s], sh), 0xFF)
        dst = pl.ds(base, bpw)
        pltpu.sync_copy(obs_v, obs_out.at[dst])
        pltpu.sync_copy(act_v, act_out.at[dst])
        pltpu.sync_copy(rew_v, rew_out.at[dst])
        pltpu.sync_copy(nobs_v, nobs_out.at[dst])
        pltpu.sync_copy(done_v, done_out.at[dst])

    return pl.kernel(
        body,
        out_type=(
            jax.ShapeDtypeStruct((B, d_obs), jnp.float32),
            jax.ShapeDtypeStruct((B, d_act), jnp.float32),
            jax.ShapeDtypeStruct((B,), jnp.float32),
            jax.ShapeDtypeStruct((B, d_obs), jnp.float32),
            jax.ShapeDtypeStruct((B,), jnp.int32),
        ),
        mesh=mesh,
        compiler_params=pltpu.CompilerParams(use_tc_tiling_on_sc=False),
        scratch_types=[
            pltpu.VMEM((bpw,), jnp.int32),
            pltpu.VMEM((bpw,), jnp.int32),
            pltpu.VMEM((bpw, d_obs), jnp.float32),
            pltpu.VMEM((bpw, d_act), jnp.float32),
            pltpu.VMEM((bpw,), jnp.float32),
            pltpu.VMEM((bpw, d_obs), jnp.float32),
            pltpu.VMEM((bpw,), jnp.int32),
            pltpu.VMEM((bpw,), jnp.int32),
            pltpu.SemaphoreType.DMA,
        ],
    )


def kernel(obs_mem, action_mem, reward_mem, next_obs_mem, done_mem, idx):
    M, d_obs = obs_mem.shape
    d_act = action_mem.shape[1]
    B = idx.shape[0]
    done_words = lax.bitcast_convert_type(
        done_mem.astype(jnp.uint8).reshape(M // 4, 4), jnp.int32)
    sampler = _build_sampler(B, d_obs, d_act)
    obs_b, act_b, rew_b, nobs_b, done_i = sampler(
        obs_mem, action_mem, reward_mem, next_obs_mem, done_words, idx)
    return obs_b, act_b, rew_b, nobs_b, done_i.astype(jnp.bool_)
